# baseline (speedup 1.0000x reference)
import functools

import jax
import jax.numpy as jnp
from jax import lax
from jax.experimental import pallas as pl
from jax.experimental.pallas import tpu as pltpu
from jax.experimental.pallas import tpu_sc as plsc


def _argmin_body(x_ref, w_ref, x2_ref, w2_ref, gf_ref, gr_ref,
                 out_ref, tf_ref, tr_ref):
    s = lax.dot_general(
        x_ref[...],
        w_ref[...],
        dimension_numbers=(((1,), (1,)), ((), ())),
        preferred_element_type=jnp.float32,
    )
    d2 = (x2_ref[...] + w2_ref[...]) - 2.0 * s

    ht = d2.shape[1]
    tmin = jnp.min(d2, axis=1, keepdims=True)
    iota = lax.broadcasted_iota(jnp.int32, d2.shape, 1)
    out_ref[...] = jnp.min(
        jnp.where(d2 == tmin, iota, ht), axis=1, keepdims=True
    )

    o, ot = gf_ref.shape[0], tf_ref.shape[1]
    v = gf_ref[...]
    if ot != o:
        v = jnp.concatenate(
            [v, jnp.zeros((ot - o, v.shape[1]), v.dtype)], axis=0
        )
    tf_ref[...] = v.T
    tr_ref[...] = gr_ref[...].T


def _tc_winners_and_tables(x, w, x2, w2, G_fwd, G_rev, o_pad, bt=512):
    b, d = x.shape
    hh = w.shape[0]
    o = G_fwd.shape[0]
    dr = G_rev.shape[0]
    nb = b // bt
    st = hh // nb
    outs = pl.pallas_call(
        _argmin_body,
        grid=(nb,),
        in_specs=[
            pl.BlockSpec((bt, d), lambda i: (i, 0)),
            pl.BlockSpec((hh, d), lambda i: (0, 0)),
            pl.BlockSpec((bt, 1), lambda i: (i, 0)),
            pl.BlockSpec((1, hh), lambda i: (0, 0)),
            pl.BlockSpec((o, st), lambda i: (0, i)),
            pl.BlockSpec((dr, st), lambda i: (0, i)),
        ],
        out_specs=[
            pl.BlockSpec((bt, 1), lambda i: (i, 0)),
            pl.BlockSpec((st, o_pad), lambda i: (i, 0)),
            pl.BlockSpec((st, dr), lambda i: (i, 0)),
        ],
        out_shape=[
            jax.ShapeDtypeStruct((b, 1), jnp.int32),
            jax.ShapeDtypeStruct((hh, o_pad), jnp.float32),
            jax.ShapeDtypeStruct((hh, dr), jnp.float32),
        ],
    )(x, w, x2, w2, G_fwd, G_rev)
    return outs


def _slice_body(x_ref, o_ref):
    o_ref[...] = x_ref[:, : o_ref.shape[1]]


def _tc_slice(x, o, bt=1024):
    b, dp = x.shape
    return pl.pallas_call(
        _slice_body,
        grid=(b // bt,),
        in_specs=[pl.BlockSpec((bt, dp), lambda i: (i, 0))],
        out_specs=pl.BlockSpec((bt, o), lambda i: (i, 0)),
        out_shape=jax.ShapeDtypeStruct((b, o), jnp.float32),
    )(x)


_NC, _NS = 2, 16
_NW = _NC * _NS


def _sc_gather_pair(tab_f, tab_r, idx, o):
    b = idx.shape[0]
    df = tab_f.shape[1]
    dr = tab_r.shape[1]
    b_per_w = b // _NW
    cf = 32
    n_chunks = b_per_w // cf
    mesh = plsc.VectorSubcoreMesh(core_axis_name="c", subcore_axis_name="s")

    @functools.partial(
        pl.kernel,
        mesh=mesh,
        out_type=[
            jax.ShapeDtypeStruct((b, df), jnp.float32),
            jax.ShapeDtypeStruct((b, dr), jnp.float32),
        ],
        scratch_types=[
            pltpu.VMEM((b_per_w,), jnp.int32),
            pltpu.VMEM((cf, df), jnp.float32),
            pltpu.VMEM((cf, df), jnp.float32),
            pltpu.VMEM((b_per_w, dr), jnp.float32),
            pltpu.SemaphoreType.DMA,
            pltpu.SemaphoreType.DMA,
            pltpu.SemaphoreType.DMA,
        ],
    )
    def k(tf_hbm, tr_hbm, idx_hbm, of_hbm, or_hbm,
          idx_v, rf0_v, rf1_v, rr_v, sem0, sem1, sem2):
        wid = lax.axis_index("s") * _NC + lax.axis_index("c")
        base = wid * b_per_w
        pltpu.sync_copy(idx_hbm.at[pl.ds(base, b_per_w)], idx_v)
        rcp = pltpu.async_copy(tr_hbm.at[idx_v], rr_v, sem2)
        bufs = (rf0_v, rf1_v)
        sems = (sem0, sem1)
        cps = [None, None]
        cps[0] = pltpu.async_copy(
            tf_hbm.at[idx_v.at[pl.ds(0, cf)]], bufs[0], sems[0]
        )
        for c in range(n_chunks):
            nxt = (c + 1) % 2
            if c + 1 < n_chunks:
                cps[nxt] = pltpu.async_copy(
                    tf_hbm.at[idx_v.at[pl.ds((c + 1) * cf, cf)]],
                    bufs[nxt],
                    sems[nxt],
                )
            cps[c % 2].wait()
            pltpu.sync_copy(bufs[c % 2], of_hbm.at[pl.ds(base + c * cf, cf)])
        rcp.wait()
        pltpu.sync_copy(rr_v, or_hbm.at[pl.ds(base, b_per_w)])

    return k(tab_f, tab_r, idx)


def kernel(x, kohonen_weights, G_fwd, G_rev):
    x = x.reshape(x.shape[0], -1)
    b = x.shape[0]
    o = G_fwd.shape[0]

    o_pad = ((o + 127) // 128) * 128
    x2 = jnp.sum(x * x, axis=1, keepdims=True)
    w2 = jnp.sum(kohonen_weights * kohonen_weights, axis=1)[None, :]

    win2d, tab_f, tab_r = _tc_winners_and_tables(
        x, kohonen_weights, x2, w2, G_fwd, G_rev, o_pad
    )
    winners = win2d.reshape(b)
    out_f, recos = _sc_gather_pair(tab_f, tab_r, winners, o)
    output = _tc_slice(out_f, o)
    return (output, recos, winners)


# speedup vs baseline: 1.1468x; 1.1468x over previous
import functools

import jax
import jax.numpy as jnp
from jax import lax
from jax.experimental import pallas as pl
from jax.experimental.pallas import tpu as pltpu
from jax.experimental.pallas import tpu_sc as plsc


def _argmin_body(x_ref, w_ref, x2_ref, w2_ref, gf_ref, gr_ref,
                 out_ref, tf_ref, tr_ref):
    s = lax.dot_general(
        x_ref[...],
        w_ref[...],
        dimension_numbers=(((1,), (1,)), ((), ())),
        preferred_element_type=jnp.float32,
    )
    d2 = (x2_ref[...] + w2_ref[...]) - 2.0 * s

    ht = d2.shape[1]
    tmin = jnp.min(d2, axis=1, keepdims=True)
    iota = lax.broadcasted_iota(jnp.int32, d2.shape, 1)
    out_ref[...] = jnp.min(
        jnp.where(d2 == tmin, iota, ht), axis=1, keepdims=True
    )

    o, ot = gf_ref.shape[0], tf_ref.shape[1]
    v = gf_ref[...]
    if ot != o:
        v = jnp.concatenate(
            [v, jnp.zeros((ot - o, v.shape[1]), v.dtype)], axis=0
        )
    tf_ref[...] = v.T
    tr_ref[...] = gr_ref[...].T


def _tc_winners_and_tables(x, w, x2, w2, G_fwd, G_rev, o_pad, bt=256):
    b, d = x.shape
    hh = w.shape[0]
    o = G_fwd.shape[0]
    dr = G_rev.shape[0]
    nb = b // bt
    st = hh // nb
    outs = pl.pallas_call(
        _argmin_body,
        grid=(nb,),
        in_specs=[
            pl.BlockSpec((bt, d), lambda i: (i, 0)),
            pl.BlockSpec((hh, d), lambda i: (0, 0)),
            pl.BlockSpec((bt, 1), lambda i: (i, 0)),
            pl.BlockSpec((1, hh), lambda i: (0, 0)),
            pl.BlockSpec((o, st), lambda i: (0, i)),
            pl.BlockSpec((dr, st), lambda i: (0, i)),
        ],
        out_specs=[
            pl.BlockSpec((bt, 1), lambda i: (i, 0)),
            pl.BlockSpec((st, o_pad), lambda i: (i, 0)),
            pl.BlockSpec((st, dr), lambda i: (i, 0)),
        ],
        out_shape=[
            jax.ShapeDtypeStruct((b, 1), jnp.int32),
            jax.ShapeDtypeStruct((hh, o_pad), jnp.float32),
            jax.ShapeDtypeStruct((hh, dr), jnp.float32),
        ],
    )(x, w, x2, w2, G_fwd, G_rev)
    return outs


_NC, _NS = 2, 16
_NW = _NC * _NS


def _sc_gather_pair(tab_f, tab_r, idx, o):
    b = idx.shape[0]
    df = tab_f.shape[1]
    dr = tab_r.shape[1]
    b_per_w = b // _NW
    cf = 32
    n_chunks = b_per_w // cf
    mesh = plsc.VectorSubcoreMesh(core_axis_name="c", subcore_axis_name="s")

    @functools.partial(
        pl.kernel,
        mesh=mesh,
        out_type=[
            jax.ShapeDtypeStruct((b, df), jnp.float32),
            jax.ShapeDtypeStruct((b, dr), jnp.float32),
        ],
        scratch_types=[
            pltpu.VMEM((b_per_w,), jnp.int32),
            pltpu.VMEM((cf, df), jnp.float32),
            pltpu.VMEM((cf, df), jnp.float32),
            pltpu.VMEM((b_per_w, dr), jnp.float32),
            pltpu.SemaphoreType.DMA,
            pltpu.SemaphoreType.DMA,
            pltpu.SemaphoreType.DMA,
        ],
    )
    def k(tf_hbm, tr_hbm, idx_hbm, of_hbm, or_hbm,
          idx_v, rf0_v, rf1_v, rr_v, sem0, sem1, sem2):
        wid = lax.axis_index("s") * _NC + lax.axis_index("c")
        base = wid * b_per_w
        pltpu.sync_copy(idx_hbm.at[pl.ds(base, b_per_w)], idx_v)
        rcp = pltpu.async_copy(tr_hbm.at[idx_v], rr_v, sem2)
        bufs = (rf0_v, rf1_v)
        sems = (sem0, sem1)
        cps = [None, None]
        cps[0] = pltpu.async_copy(
            tf_hbm.at[idx_v.at[pl.ds(0, cf)]], bufs[0], sems[0]
        )
        for c in range(n_chunks):
            nxt = (c + 1) % 2
            if c + 1 < n_chunks:
                cps[nxt] = pltpu.async_copy(
                    tf_hbm.at[idx_v.at[pl.ds((c + 1) * cf, cf)]],
                    bufs[nxt],
                    sems[nxt],
                )
            cps[c % 2].wait()
            pltpu.sync_copy(bufs[c % 2], of_hbm.at[pl.ds(base + c * cf, cf)])
        rcp.wait()
        pltpu.sync_copy(rr_v, or_hbm.at[pl.ds(base, b_per_w)])

    return k(tab_f, tab_r, idx)


def kernel(x, kohonen_weights, G_fwd, G_rev):
    x = x.reshape(x.shape[0], -1)
    b = x.shape[0]
    o = G_fwd.shape[0]

    o_pad = ((o + 127) // 128) * 128
    x2 = jnp.sum(x * x, axis=1, keepdims=True)
    w2 = jnp.sum(kohonen_weights * kohonen_weights, axis=1)[None, :]

    win2d, tab_f, tab_r = _tc_winners_and_tables(
        x, kohonen_weights, x2, w2, G_fwd, G_rev, o_pad
    )
    winners = win2d.reshape(b)
    out_f, recos = _sc_gather_pair(tab_f, tab_r, winners, o)
    output = out_f[:, :o]
    return (output, recos, winners)


# speedup vs baseline: 1.1584x; 1.0101x over previous
import functools

import jax
import jax.numpy as jnp
from jax import lax
from jax.experimental import pallas as pl
from jax.experimental.pallas import tpu as pltpu
from jax.experimental.pallas import tpu_sc as plsc


def _argmin_body(x_ref, w_ref, x2_ref, w2_ref, gf_ref, gr_ref,
                 out_ref, tf_ref, tr_ref):
    s = lax.dot_general(
        x_ref[...],
        w_ref[...],
        dimension_numbers=(((1,), (1,)), ((), ())),
        preferred_element_type=jnp.float32,
    )
    d2 = (x2_ref[...] + w2_ref[...]) - 2.0 * s

    ht = d2.shape[1]
    tmin = jnp.min(d2, axis=1, keepdims=True)
    iota = lax.broadcasted_iota(jnp.int32, d2.shape, 1)
    out_ref[...] = jnp.min(jnp.where(d2 == tmin, iota, ht), axis=1)

    o, ot = gf_ref.shape[0], tf_ref.shape[1]
    v = gf_ref[...]
    if ot != o:
        v = jnp.concatenate(
            [v, jnp.zeros((ot - o, v.shape[1]), v.dtype)], axis=0
        )
    tf_ref[...] = v.T
    tr_ref[...] = gr_ref[...].T


def _tc_winners_and_tables(x, w, x2, w2, G_fwd, G_rev, o_pad, bt=256):
    b, d = x.shape
    hh = w.shape[0]
    o = G_fwd.shape[0]
    dr = G_rev.shape[0]
    nb = b // bt
    st = hh // nb
    outs = pl.pallas_call(
        _argmin_body,
        grid=(nb,),
        in_specs=[
            pl.BlockSpec((bt, d), lambda i: (i, 0)),
            pl.BlockSpec((hh, d), lambda i: (0, 0)),
            pl.BlockSpec((bt, 1), lambda i: (i, 0)),
            pl.BlockSpec((1, hh), lambda i: (0, 0)),
            pl.BlockSpec((o, st), lambda i: (0, i)),
            pl.BlockSpec((dr, st), lambda i: (0, i)),
        ],
        out_specs=[
            pl.BlockSpec((bt,), lambda i: (i,)),
            pl.BlockSpec((st, o_pad), lambda i: (i, 0)),
            pl.BlockSpec((st, dr), lambda i: (i, 0)),
        ],
        out_shape=[
            jax.ShapeDtypeStruct((b,), jnp.int32),
            jax.ShapeDtypeStruct((hh, o_pad), jnp.float32),
            jax.ShapeDtypeStruct((hh, dr), jnp.float32),
        ],
    )(x, w, x2, w2, G_fwd, G_rev)
    return outs


_NC, _NS = 2, 16
_NW = _NC * _NS


def _sc_gather_pair(tab_f, tab_r, idx, o):
    b = idx.shape[0]
    df = tab_f.shape[1]
    dr = tab_r.shape[1]
    b_per_w = b // _NW
    cf = 32
    n_chunks = b_per_w // cf
    mesh = plsc.VectorSubcoreMesh(core_axis_name="c", subcore_axis_name="s")

    @functools.partial(
        pl.kernel,
        mesh=mesh,
        out_type=[
            jax.ShapeDtypeStruct((b, df), jnp.float32),
            jax.ShapeDtypeStruct((b, dr), jnp.float32),
        ],
        scratch_types=[
            pltpu.VMEM((b_per_w,), jnp.int32),
            pltpu.VMEM((cf, df), jnp.float32),
            pltpu.VMEM((cf, df), jnp.float32),
            pltpu.VMEM((b_per_w, dr), jnp.float32),
            pltpu.SemaphoreType.DMA,
            pltpu.SemaphoreType.DMA,
            pltpu.SemaphoreType.DMA,
        ],
    )
    def k(tf_hbm, tr_hbm, idx_hbm, of_hbm, or_hbm,
          idx_v, rf0_v, rf1_v, rr_v, sem0, sem1, sem2):
        wid = lax.axis_index("s") * _NC + lax.axis_index("c")
        base = wid * b_per_w
        pltpu.sync_copy(idx_hbm.at[pl.ds(base, b_per_w)], idx_v)
        rcp = pltpu.async_copy(tr_hbm.at[idx_v], rr_v, sem2)
        bufs = (rf0_v, rf1_v)
        sems = (sem0, sem1)
        cps = [None, None]
        cps[0] = pltpu.async_copy(
            tf_hbm.at[idx_v.at[pl.ds(0, cf)]], bufs[0], sems[0]
        )
        for c in range(n_chunks):
            nxt = (c + 1) % 2
            if c + 1 < n_chunks:
                cps[nxt] = pltpu.async_copy(
                    tf_hbm.at[idx_v.at[pl.ds((c + 1) * cf, cf)]],
                    bufs[nxt],
                    sems[nxt],
                )
            cps[c % 2].wait()
            pltpu.sync_copy(bufs[c % 2], of_hbm.at[pl.ds(base + c * cf, cf)])
        rcp.wait()
        pltpu.sync_copy(rr_v, or_hbm.at[pl.ds(base, b_per_w)])

    return k(tab_f, tab_r, idx)


def kernel(x, kohonen_weights, G_fwd, G_rev):
    x = x.reshape(x.shape[0], -1)
    b = x.shape[0]
    o = G_fwd.shape[0]

    o_pad = ((o + 127) // 128) * 128
    x2 = jnp.sum(x * x, axis=1, keepdims=True)
    w2 = jnp.sum(kohonen_weights * kohonen_weights, axis=1)[None, :]

    winners, tab_f, tab_r = _tc_winners_and_tables(
        x, kohonen_weights, x2, w2, G_fwd, G_rev, o_pad
    )
    out_f, recos = _sc_gather_pair(tab_f, tab_r, winners, o)
    output = out_f[:, :o]
    return (output, recos, winners)
